# baseline (speedup 1.0000x reference)
import functools

import jax
import jax.numpy as jnp
from jax import lax
from jax.experimental import pallas as pl
from jax.experimental.pallas import tpu as pltpu
from jax.experimental.pallas import tpu_sc as plsc

_D = 64
_NB = 4096 * 200
_NC, _NS = 2, 16
_NW = _NC * _NS
_BPW = _NB // _NW
_C = 128
_NCHUNK = _BPW // _C

_mesh = plsc.VectorSubcoreMesh(core_axis_name="c", subcore_axis_name="s")


@functools.partial(
    pl.kernel,
    out_type=jax.ShapeDtypeStruct((_NB, _D), jnp.float32),
    mesh=_mesh,
    scratch_types=[
        pltpu.VMEM((_BPW,), jnp.int32),
        pltpu.VMEM((_C, _D), jnp.float32),
        pltpu.SemaphoreType.DMA,
    ],
    compiler_params=pltpu.CompilerParams(use_tc_tiling_on_sc=False),
)
def _gather_kernel(idx_hbm, table_hbm, out_hbm, idx_v, rows_v, sem):
    wid = lax.axis_index("s") * _NC + lax.axis_index("c")
    base = wid * _BPW
    pltpu.sync_copy(idx_hbm.at[pl.ds(base, _BPW)], idx_v)

    def body(c, carry):
        off = c * _C
        pltpu.async_copy(
            table_hbm.at[idx_v.at[pl.ds(off, _C)]], rows_v, sem
        ).wait()
        pltpu.sync_copy(rows_v, out_hbm.at[pl.ds(base + off, _C)])
        return carry

    lax.fori_loop(0, _NCHUNK, body, 0)


def kernel(x, table):
    idx = x.reshape(-1)
    out = _gather_kernel(idx, table)
    return out.reshape(x.shape + (table.shape[1],))


# speedup vs baseline: 1.1153x; 1.1153x over previous
import functools

import jax
import jax.numpy as jnp
from jax import lax
from jax.experimental import pallas as pl
from jax.experimental.pallas import tpu as pltpu
from jax.experimental.pallas import tpu_sc as plsc

_D = 64
_NB = 4096 * 200
_NC, _NS = 2, 16
_NW = _NC * _NS
_BPW = _NB // _NW
_C = 128
_NBUF = 8
_NCHUNK = _BPW // _C
_NROUNDS = _NCHUNK // _NBUF

_mesh = plsc.VectorSubcoreMesh(core_axis_name="c", subcore_axis_name="s")


@functools.partial(
    pl.kernel,
    out_type=jax.ShapeDtypeStruct((_NB, _D), jnp.float32),
    mesh=_mesh,
    scratch_types=[
        pltpu.VMEM((_BPW,), jnp.int32),
        pltpu.VMEM((_NBUF, _C, _D), jnp.float32),
        pltpu.SemaphoreType.DMA((_NBUF,)),
        pltpu.SemaphoreType.DMA((_NBUF,)),
    ],
    compiler_params=pltpu.CompilerParams(use_tc_tiling_on_sc=False),
)
def _gather_kernel(idx_hbm, table_hbm, out_hbm, idx_v, rows_v, sem_g, sem_s):
    wid = lax.axis_index("s") * _NC + lax.axis_index("c")
    base = wid * _BPW
    pltpu.sync_copy(idx_hbm.at[pl.ds(base, _BPW)], idx_v)

    def g_desc(c, b):
        return pltpu.make_async_copy(
            table_hbm.at[idx_v.at[pl.ds(c * _C, _C)]], rows_v.at[b],
            sem_g.at[b])

    def s_desc(c, b):
        return pltpu.make_async_copy(
            rows_v.at[b], out_hbm.at[pl.ds(base + c * _C, _C)], sem_s.at[b])

    for b in range(_NBUF):
        g_desc(b, b).start()

    def round_body(r, carry):
        c0 = r * _NBUF
        for b in range(_NBUF):
            g_desc(c0 + b, b).wait()
            s_desc(c0 + b, b).start()
        for b in range(_NBUF):
            s_desc(c0 + b, b).wait()
            g_desc(c0 + _NBUF + b, b).start()
        return carry

    lax.fori_loop(0, _NROUNDS - 1, round_body, 0)

    c0 = (_NROUNDS - 1) * _NBUF
    for b in range(_NBUF):
        g_desc(c0 + b, b).wait()
        s_desc(c0 + b, b).start()
    for b in range(_NBUF):
        s_desc(c0 + b, b).wait()


def kernel(x, table):
    idx = x.reshape(-1)
    out = _gather_kernel(idx, table)
    return out.reshape(x.shape + (table.shape[1],))
